# baseline (speedup 1.0000x reference)
import numpy as np
import jax
import jax.numpy as jnp
from jax import lax
from jax.experimental import pallas as pl
from jax.experimental.pallas import tpu as pltpu
from jax.experimental.pallas import tpu_sc as plsc

_B, _C, _H, _W = 8, 96, 224, 224
_N = _H * _W
_R = _B * _C
_K = int(0.3 * _N)
_M31 = np.int32(0x7FFFFFFF)
_CAP = 16384
_NT = (_H // 8) * 2


def _f2key(v):
    u = plsc.bitcast(v, jnp.int32)
    return u ^ ((u >> 31) & _M31)


def _popcount(m):
    return jnp.max(plsc.all_reduce_population_count(m))


def _walk(histref, nvec, kr):
    lanes = lax.iota(jnp.int32, 16)

    def body(j, carry):
        found, dstar, newk, running = carry
        jj = nvec - 1 - j
        h = histref[pl.ds(jj * 16, 16)]
        suf = lax.rev(jnp.cumsum(lax.rev(h, (0,)), axis=0), (0,)) + running
        mask = suf >= kr
        c = _popcount(mask)
        has = jnp.logical_and(found == 0, c > 0)
        sel = lanes == (c - 1)
        s_d = jnp.max(jnp.where(sel, suf, 0))
        h_d = jnp.max(jnp.where(sel, h, 0))
        dstar = jnp.where(has, jj * 16 + c - 1, dstar)
        newk = jnp.where(has, kr - (s_d - h_d), newk)
        found = jnp.where(has, jnp.int32(1), found)
        running = jnp.max(suf)
        return found, dstar, newk, running

    z = jnp.int32(0)
    _, dstar, newk, _ = lax.fori_loop(0, nvec, body, (z, z, z, z))
    return dstar, newk


def _find_hier(hist, histc, nb, kr):
    lanes = lax.iota(jnp.int32, 16)
    dc, kr2 = _walk(histc, nb // 256, kr)
    h = hist[pl.ds(dc * 16, 16)]
    suf = lax.rev(jnp.cumsum(lax.rev(h, (0,)), axis=0), (0,))
    mask = suf >= kr2
    c = _popcount(mask)
    sel = lanes == (c - 1)
    s_d = jnp.max(jnp.where(sel, suf, 0))
    h_d = jnp.max(jnp.where(sel, h, 0))
    return dc * 16 + c - 1, kr2 - (s_d - h_d)


def _zero_hist(hist, histc, nb):
    zeros16 = jnp.zeros((16,), jnp.int32)

    @plsc.parallel_loop(0, nb, 16, unroll=4)
    def _(i):
        hist[pl.ds(i, 16)] = zeros16

    @plsc.parallel_loop(0, nb // 16, 16, unroll=1)
    def _(i):
        histc[pl.ds(i, 16)] = zeros16


def _build_coarse(hist, histc, nb):
    @plsc.parallel_loop(0, nb // 16, 1, unroll=4)
    def _(i):
        idx = jnp.full((16,), i, jnp.int32)
        plsc.addupdate_scatter(histc, [idx], hist[pl.ds(i * 16, 16)])


def _slices(dataA, dataB, rr2):
    tr = rr2 >> 3
    rs = rr2 & 7
    out = [(dataA, tr, rs, c * 16) for c in range(8)]
    out += [(dataB, tr, rs, c * 16) for c in range(6)]
    return out


def _row_threshold(dataA, dataB, cand, hist, histc):
    lanes = lax.iota(jnp.int32, 16)
    ones16 = jnp.ones((16,), jnp.int32)
    zi16 = jnp.zeros((16,), jnp.int32)

    _zero_hist(hist, histc, 4096)

    @plsc.parallel_loop(0, _H, 1, unroll=2)
    def _(r):
        for (ref, tr, rs, c) in _slices(dataA, dataB, r):
            key = _f2key(ref[tr, rs, pl.ds(c, 16)])
            d = (key >> 20) + 2048
            plsc.addupdate_scatter(hist, [d], ones16)

    _build_coarse(hist, histc, 4096)
    d1, kr1 = _find_hier(hist, histc, 4096, jnp.int32(_K))

    @plsc.parallel_loop(0, _H, 1, unroll=2, carry=zi16)
    def c1_off(r, off):
        for (ref, tr, rs, c) in _slices(dataA, dataB, r):
            key = _f2key(ref[tr, rs, pl.ds(c, 16)])
            m = ((key >> 20) + 2048) == d1
            mi = m.astype(jnp.int32)
            idx = off + jnp.cumsum(mi, axis=0) - mi
            m_w = jnp.logical_and(m, idx < _CAP)
            plsc.store_scatter(cand, [jnp.minimum(idx, _CAP - 1)], key, mask=m_w)
            off = off + plsc.all_reduce_population_count(m)
        return off

    m1 = jnp.max(c1_off)

    def small_levels():
        ns1 = (m1 + 15) >> 4
        _zero_hist(hist, histc, 1024)

        def h2(i, _):
            kv = cand[pl.ds(i * 16, 16)]
            valid = (i * 16 + lanes) < m1
            d = (kv >> 10) & jnp.int32(0x3FF)
            plsc.addupdate_scatter(hist, [d], ones16, mask=valid)
            return 0

        lax.fori_loop(0, ns1, h2, 0)
        _build_coarse(hist, histc, 1024)
        d2, kr2 = _find_hier(hist, histc, 1024, kr1)

        def c2(i, off):
            kv = cand[pl.ds(i * 16, 16)]
            valid = (i * 16 + lanes) < m1
            m = jnp.logical_and(((kv >> 10) & jnp.int32(0x3FF)) == d2, valid)
            plsc.store_compressed(cand.at[pl.ds(off, 16)], kv, mask=m)
            return off + _popcount(m)

        m2 = lax.fori_loop(0, ns1, c2, jnp.int32(0))
        ns2 = (m2 + 15) >> 4

        _zero_hist(hist, histc, 1024)

        def h3(i, _):
            kv = cand[pl.ds(i * 16, 16)]
            valid = (i * 16 + lanes) < m2
            d = kv & jnp.int32(0x3FF)
            plsc.addupdate_scatter(hist, [d], ones16, mask=valid)
            return 0

        lax.fori_loop(0, ns2, h3, 0)
        _build_coarse(hist, histc, 1024)
        d3, _ = _find_hier(hist, histc, 1024, kr2)
        return (d2 << 10) | d3

    def big_levels():
        _zero_hist(hist, histc, 1024)

        @plsc.parallel_loop(0, _H, 1, unroll=2)
        def _(r):
            for (ref, tr, rs, c) in _slices(dataA, dataB, r):
                key = _f2key(ref[tr, rs, pl.ds(c, 16)])
                v1 = ((key >> 20) + 2048) == d1
                d = (key >> 10) & jnp.int32(0x3FF)
                plsc.addupdate_scatter(hist, [d], ones16, mask=v1)

        _build_coarse(hist, histc, 1024)
        d2, kr2 = _find_hier(hist, histc, 1024, kr1)
        _zero_hist(hist, histc, 1024)

        @plsc.parallel_loop(0, _H, 1, unroll=2)
        def _(r):
            for (ref, tr, rs, c) in _slices(dataA, dataB, r):
                key = _f2key(ref[tr, rs, pl.ds(c, 16)])
                m = jnp.logical_and(((key >> 20) + 2048) == d1,
                                    ((key >> 10) & jnp.int32(0x3FF)) == d2)
                d = key & jnp.int32(0x3FF)
                plsc.addupdate_scatter(hist, [d], ones16, mask=m)

        _build_coarse(hist, histc, 1024)
        d3, _ = _find_hier(hist, histc, 1024, kr2)
        return (d2 << 10) | d3

    low20 = lax.cond(m1 <= _CAP, small_levels, big_levels)
    thr_key = ((d1 - 2048) << 20) | low20
    tk = jnp.full((16,), thr_key, jnp.int32)
    return plsc.bitcast(tk ^ ((tk >> 31) & _M31), jnp.float32)


def _dma_map(x_hbm, b0, ch, dataA, dataB, sem, to_vmem):
    for phase in range(2):
        for tr in range(_H // 8):
            for (ref, cs, cw) in ((dataA, 0, 128), (dataB, 128, 96)):
                hslice = x_hbm.at[b0, ch, pl.ds(tr * 8, 8), pl.ds(cs, cw)]
                vslice = ref.at[tr]
                src, dst = (hslice, vslice) if to_vmem else (vslice, hslice)
                if phase == 0:
                    pltpu.async_copy(src, dst, sem)
                else:
                    pltpu.make_async_copy(src, dst, sem).wait()


def _sc_body(x_hbm, out_hbm, dataA, dataB, cand, hist, histc, sem):
    nc = 2
    rpw = _R // (nc * 16)
    wid = lax.axis_index("s") * nc + lax.axis_index("c")
    b0 = wid // 4
    c0 = (wid % 4) * rpw
    zf16 = jnp.zeros((16,), jnp.float32)

    def row_body(rr, _):
        ch = c0 + rr
        _dma_map(x_hbm, b0, ch, dataA, dataB, sem, True)

        thr = _row_threshold(dataA, dataB, cand, hist, histc)

        @plsc.parallel_loop(0, _H, 1, unroll=2)
        def _(r):
            for (ref, tr, rs, c) in _slices(dataA, dataB, r):
                v = ref[tr, rs, pl.ds(c, 16)]
                ref[tr, rs, pl.ds(c, 16)] = jnp.where(v >= thr, v, zf16)

        _dma_map(out_hbm, b0, ch, dataA, dataB, sem, False)
        return 0

    lax.fori_loop(0, rpw, row_body, 0)


def _build():
    mesh = plsc.VectorSubcoreMesh(core_axis_name="c", subcore_axis_name="s")
    return pl.kernel(
        _sc_body,
        out_type=jax.ShapeDtypeStruct((_B, _C, _H, _W), jnp.float32),
        mesh=mesh,
        scratch_types=[
            pltpu.VMEM((_H // 8, 8, 128), jnp.float32),
            pltpu.VMEM((_H // 8, 8, 96), jnp.float32),
            pltpu.VMEM((_CAP,), jnp.int32),
            pltpu.VMEM((4096,), jnp.int32),
            pltpu.VMEM((256,), jnp.int32),
            pltpu.SemaphoreType.DMA,
        ],
        compiler_params=pltpu.CompilerParams(
            needs_layout_passes=False,
            use_tc_tiling_on_sc=True,
        ),
    )


def kernel(x):
    return _build()(x)


# speedup vs baseline: 2.2867x; 2.2867x over previous
import numpy as np
import jax
import jax.numpy as jnp
from jax import lax
from jax.experimental import pallas as pl
from jax.experimental.pallas import tpu as pltpu
from jax.experimental.pallas import tpu_sc as plsc

_B, _C, _H, _W = 8, 96, 224, 224
_N = _H * _W
_R = _B * _C
_K = int(0.3 * _N)
_M31 = np.int32(0x7FFFFFFF)
_NT = (_H // 8) * 2


def _f2key(v):
    u = plsc.bitcast(v, jnp.int32)
    return u ^ ((u >> 31) & _M31)


def _popcount(m):
    return jnp.max(plsc.all_reduce_population_count(m))


def _walk(histref, nvec, kr):
    lanes = lax.iota(jnp.int32, 16)

    def body(j, carry):
        found, dstar, newk, running = carry
        jj = nvec - 1 - j
        h = histref[pl.ds(jj * 16, 16)]
        suf = lax.rev(jnp.cumsum(lax.rev(h, (0,)), axis=0), (0,)) + running
        mask = suf >= kr
        c = _popcount(mask)
        has = jnp.logical_and(found == 0, c > 0)
        sel = lanes == (c - 1)
        s_d = jnp.max(jnp.where(sel, suf, 0))
        h_d = jnp.max(jnp.where(sel, h, 0))
        dstar = jnp.where(has, jj * 16 + c - 1, dstar)
        newk = jnp.where(has, kr - (s_d - h_d), newk)
        found = jnp.where(has, jnp.int32(1), found)
        running = jnp.max(suf)
        return found, dstar, newk, running

    z = jnp.int32(0)
    _, dstar, newk, _ = lax.fori_loop(0, nvec, body, (z, z, z, z))
    return dstar, newk


def _find_hier(hist, histc, nb, kr):
    lanes = lax.iota(jnp.int32, 16)
    dc, kr2 = _walk(histc, nb // 256, kr)
    h = hist[pl.ds(dc * 16, 16)]
    suf = lax.rev(jnp.cumsum(lax.rev(h, (0,)), axis=0), (0,))
    mask = suf >= kr2
    c = _popcount(mask)
    sel = lanes == (c - 1)
    s_d = jnp.max(jnp.where(sel, suf, 0))
    h_d = jnp.max(jnp.where(sel, h, 0))
    return dc * 16 + c - 1, kr2 - (s_d - h_d)


def _zero_hist(hist, histc, nb):
    zeros16 = jnp.zeros((16,), jnp.int32)

    @plsc.parallel_loop(0, nb, 16, unroll=4)
    def _(i):
        hist[pl.ds(i, 16)] = zeros16

    @plsc.parallel_loop(0, nb // 16, 16, unroll=1)
    def _(i):
        histc[pl.ds(i, 16)] = zeros16


def _build_coarse(hist, histc, nb):
    @plsc.parallel_loop(0, nb // 16, 1, unroll=4)
    def _(i):
        idx = jnp.full((16,), i, jnp.int32)
        plsc.addupdate_scatter(histc, [idx], hist[pl.ds(i * 16, 16)])


def _slices(dataA, dataB, rr2):
    tr = rr2 >> 3
    rs = rr2 & 7
    out = [(dataA, tr, rs, c * 16) for c in range(8)]
    out += [(dataB, tr, rs, c * 16) for c in range(6)]
    return out


def _row_threshold(dataA, dataB, hist, histc):
    lanes = lax.iota(jnp.int32, 16)
    ones16 = jnp.ones((16,), jnp.int32)

    _zero_hist(hist, histc, 4096)

    @plsc.parallel_loop(0, _H, 1, unroll=2)
    def _(r):
        for (ref, tr, rs, c) in _slices(dataA, dataB, r):
            key = _f2key(ref[tr, rs, pl.ds(c, 16)])
            d = (key >> 20) + 2048
            plsc.addupdate_scatter(hist, [d], ones16)

    _build_coarse(hist, histc, 4096)
    d1, kr1 = _find_hier(hist, histc, 4096, jnp.int32(_K))

    _zero_hist(hist, histc, 1024)

    @plsc.parallel_loop(0, _H, 1, unroll=2)
    def _(r):
        for (ref, tr, rs, c) in _slices(dataA, dataB, r):
            key = _f2key(ref[tr, rs, pl.ds(c, 16)])
            v1 = ((key >> 20) + 2048) == d1
            d = (key >> 10) & jnp.int32(0x3FF)
            plsc.addupdate_scatter(hist, [d], ones16, mask=v1)

    _build_coarse(hist, histc, 1024)
    d2, kr2 = _find_hier(hist, histc, 1024, kr1)

    _zero_hist(hist, histc, 1024)

    @plsc.parallel_loop(0, _H, 1, unroll=2)
    def _(r):
        for (ref, tr, rs, c) in _slices(dataA, dataB, r):
            key = _f2key(ref[tr, rs, pl.ds(c, 16)])
            m = jnp.logical_and(((key >> 20) + 2048) == d1,
                                ((key >> 10) & jnp.int32(0x3FF)) == d2)
            d = key & jnp.int32(0x3FF)
            plsc.addupdate_scatter(hist, [d], ones16, mask=m)

    _build_coarse(hist, histc, 1024)
    d3, _ = _find_hier(hist, histc, 1024, kr2)
    low20 = (d2 << 10) | d3

    thr_key = ((d1 - 2048) << 20) | low20
    tk = jnp.full((16,), thr_key, jnp.int32)
    return plsc.bitcast(tk ^ ((tk >> 31) & _M31), jnp.float32)


def _dma_map(x_hbm, b0, ch, dataA, dataB, sem, to_vmem):
    for phase in range(2):
        for tr in range(_H // 8):
            for (ref, cs, cw) in ((dataA, 0, 128), (dataB, 128, 96)):
                hslice = x_hbm.at[b0, ch, pl.ds(tr * 8, 8), pl.ds(cs, cw)]
                vslice = ref.at[tr]
                src, dst = (hslice, vslice) if to_vmem else (vslice, hslice)
                if phase == 0:
                    pltpu.async_copy(src, dst, sem)
                else:
                    pltpu.make_async_copy(src, dst, sem).wait()


def _sc_body(x_hbm, out_hbm, dataA, dataB, hist, histc, sem):
    nc = 2
    rpw = _R // (nc * 16)
    wid = lax.axis_index("s") * nc + lax.axis_index("c")
    b0 = wid // 4
    c0 = (wid % 4) * rpw
    zf16 = jnp.zeros((16,), jnp.float32)

    def row_body(rr, _):
        ch = c0 + rr
        _dma_map(x_hbm, b0, ch, dataA, dataB, sem, True)

        thr = _row_threshold(dataA, dataB, hist, histc)

        @plsc.parallel_loop(0, _H, 1, unroll=2)
        def _(r):
            for (ref, tr, rs, c) in _slices(dataA, dataB, r):
                v = ref[tr, rs, pl.ds(c, 16)]
                ref[tr, rs, pl.ds(c, 16)] = jnp.where(v >= thr, v, zf16)

        _dma_map(out_hbm, b0, ch, dataA, dataB, sem, False)
        return 0

    lax.fori_loop(0, rpw, row_body, 0)


def _build():
    mesh = plsc.VectorSubcoreMesh(core_axis_name="c", subcore_axis_name="s")
    return pl.kernel(
        _sc_body,
        out_type=jax.ShapeDtypeStruct((_B, _C, _H, _W), jnp.float32),
        mesh=mesh,
        scratch_types=[
            pltpu.VMEM((_H // 8, 8, 128), jnp.float32),
            pltpu.VMEM((_H // 8, 8, 96), jnp.float32),
            pltpu.VMEM((4096,), jnp.int32),
            pltpu.VMEM((256,), jnp.int32),
            pltpu.SemaphoreType.DMA,
        ],
        compiler_params=pltpu.CompilerParams(
            needs_layout_passes=False,
            use_tc_tiling_on_sc=True,
        ),
    )


def kernel(x):
    return _build()(x)


# speedup vs baseline: 2.5685x; 1.1232x over previous
import numpy as np
import jax
import jax.numpy as jnp
from jax import lax
from jax.experimental import pallas as pl
from jax.experimental.pallas import tpu as pltpu
from jax.experimental.pallas import tpu_sc as plsc

_B, _C, _H, _W = 8, 96, 224, 224
_N = _H * _W
_R = _B * _C
_K = int(0.3 * _N)
_M31 = np.int32(0x7FFFFFFF)
_NT = (_H // 8) * 2


def _f2key(v):
    u = plsc.bitcast(v, jnp.int32)
    return u ^ ((u >> 31) & _M31)


def _popcount(m):
    return jnp.max(plsc.all_reduce_population_count(m))


def _walk(histref, nvec, kr):
    lanes = lax.iota(jnp.int32, 16)

    def body(j, carry):
        found, dstar, newk, running = carry
        jj = nvec - 1 - j
        h = histref[pl.ds(jj * 16, 16)]
        suf = lax.rev(jnp.cumsum(lax.rev(h, (0,)), axis=0), (0,)) + running
        mask = suf >= kr
        c = _popcount(mask)
        has = jnp.logical_and(found == 0, c > 0)
        sel = lanes == (c - 1)
        s_d = jnp.max(jnp.where(sel, suf, 0))
        h_d = jnp.max(jnp.where(sel, h, 0))
        dstar = jnp.where(has, jj * 16 + c - 1, dstar)
        newk = jnp.where(has, kr - (s_d - h_d), newk)
        found = jnp.where(has, jnp.int32(1), found)
        running = jnp.max(suf)
        return found, dstar, newk, running

    z = jnp.int32(0)
    _, dstar, newk, _ = lax.fori_loop(0, nvec, body, (z, z, z, z))
    return dstar, newk


def _find_hier(hist, histc, nb, kr):
    lanes = lax.iota(jnp.int32, 16)
    dc, kr2 = _walk(histc, nb // 256, kr)
    h = hist[pl.ds(dc * 16, 16)]
    suf = lax.rev(jnp.cumsum(lax.rev(h, (0,)), axis=0), (0,))
    mask = suf >= kr2
    c = _popcount(mask)
    sel = lanes == (c - 1)
    s_d = jnp.max(jnp.where(sel, suf, 0))
    h_d = jnp.max(jnp.where(sel, h, 0))
    return dc * 16 + c - 1, kr2 - (s_d - h_d)


def _zero_hist(hist, histc, nb):
    zeros16 = jnp.zeros((16,), jnp.int32)

    @plsc.parallel_loop(0, nb, 16, unroll=4)
    def _(i):
        hist[pl.ds(i, 16)] = zeros16

    @plsc.parallel_loop(0, nb // 16, 16, unroll=1)
    def _(i):
        histc[pl.ds(i, 16)] = zeros16


def _build_coarse(hist, histc, nb):
    @plsc.parallel_loop(0, nb // 16, 1, unroll=4)
    def _(i):
        idx = jnp.full((16,), i, jnp.int32)
        plsc.addupdate_scatter(histc, [idx], hist[pl.ds(i * 16, 16)])


def _slices(dataA, dataB, rr2):
    tr = rr2 >> 3
    rs = rr2 & 7
    out = [(dataA, tr, rs, c * 16) for c in range(8)]
    out += [(dataB, tr, rs, c * 16) for c in range(6)]
    return out


def _row_threshold(dataA, dataB, hist, histc):
    lanes = lax.iota(jnp.int32, 16)
    ones16 = jnp.ones((16,), jnp.int32)

    _zero_hist(hist, histc, 4096)

    @plsc.parallel_loop(0, _H, 1, unroll=2)
    def _(r):
        for (ref, tr, rs, c) in _slices(dataA, dataB, r):
            key = _f2key(ref[tr, rs, pl.ds(c, 16)])
            d = (key >> 20) + 2048
            plsc.addupdate_scatter(hist, [d], ones16)

    _build_coarse(hist, histc, 4096)
    d1, kr1 = _find_hier(hist, histc, 4096, jnp.int32(_K))

    _zero_hist(hist, histc, 1024)

    @plsc.parallel_loop(0, _H, 1, unroll=2)
    def _(r):
        for (ref, tr, rs, c) in _slices(dataA, dataB, r):
            key = _f2key(ref[tr, rs, pl.ds(c, 16)])
            v1 = ((key >> 20) + 2048) == d1
            d = (key >> 10) & jnp.int32(0x3FF)
            plsc.addupdate_scatter(hist, [d], ones16, mask=v1)

    _build_coarse(hist, histc, 1024)
    d2, kr2 = _find_hier(hist, histc, 1024, kr1)

    _zero_hist(hist, histc, 1024)

    @plsc.parallel_loop(0, _H, 1, unroll=2)
    def _(r):
        for (ref, tr, rs, c) in _slices(dataA, dataB, r):
            key = _f2key(ref[tr, rs, pl.ds(c, 16)])
            m = jnp.logical_and(((key >> 20) + 2048) == d1,
                                ((key >> 10) & jnp.int32(0x3FF)) == d2)
            d = key & jnp.int32(0x3FF)
            plsc.addupdate_scatter(hist, [d], ones16, mask=m)

    _build_coarse(hist, histc, 1024)
    d3, _ = _find_hier(hist, histc, 1024, kr2)
    low20 = (d2 << 10) | d3

    thr_key = ((d1 - 2048) << 20) | low20
    tk = jnp.full((16,), thr_key, jnp.int32)
    return plsc.bitcast(tk ^ ((tk >> 31) & _M31), jnp.float32)


def _dma_map(x_hbm, b0, ch, dataA, dataB, sem, to_vmem, fire=True, drain=True):
    phases = ([0] if fire else []) + ([1] if drain else [])
    for phase in phases:
        for tr in range(_H // 8):
            for (ref, cs, cw) in ((dataA, 0, 128), (dataB, 128, 96)):
                hslice = x_hbm.at[b0, ch, pl.ds(tr * 8, 8), pl.ds(cs, cw)]
                vslice = ref.at[tr]
                src, dst = (hslice, vslice) if to_vmem else (vslice, hslice)
                if phase == 0:
                    pltpu.async_copy(src, dst, sem)
                else:
                    pltpu.make_async_copy(src, dst, sem).wait()


def _sc_body(x_hbm, out_hbm, dataA0, dataB0, dataA1, dataB1,
             hist, histc, isem0, isem1, osem0, osem1):
    nc = 2
    rpw = _R // (nc * 16)
    wid = lax.axis_index("s") * nc + lax.axis_index("c")
    b0 = wid // 4
    c0 = (wid % 4) * rpw
    zf16 = jnp.zeros((16,), jnp.float32)
    bufs = ((dataA0, dataB0), (dataA1, dataB1))
    isems = (isem0, isem1)
    osems = (osem0, osem1)

    _dma_map(x_hbm, b0, c0, dataA0, dataB0, isem0, True, drain=False)

    def pair_body(p, _):
        for b in range(2):
            rr = 2 * p + b
            ch = c0 + rr
            dA, dB = bufs[b]
            dAo, dBo = bufs[1 - b]

            _dma_map(x_hbm, b0, ch, dA, dB, isems[b], True, fire=False)

            thr = _row_threshold(dA, dB, hist, histc)

            @pl.when(rr > 0)
            def _():
                _dma_map(out_hbm, b0, ch - 1, dAo, dBo, osems[1 - b], False,
                         fire=False)

            @pl.when(rr + 1 < rpw)
            def _():
                _dma_map(x_hbm, b0, ch + 1, dAo, dBo, isems[1 - b], True,
                         drain=False)

            @plsc.parallel_loop(0, _H, 1, unroll=2)
            def _(r):
                for (ref, tr, rs, c) in _slices(dA, dB, r):
                    v = ref[tr, rs, pl.ds(c, 16)]
                    ref[tr, rs, pl.ds(c, 16)] = jnp.where(v >= thr, v, zf16)

            _dma_map(out_hbm, b0, ch, dA, dB, osems[b], False, drain=False)
        return 0

    lax.fori_loop(0, rpw // 2, pair_body, 0)
    _dma_map(out_hbm, b0, c0 + rpw - 1, dataA1, dataB1, osems[1], False,
             fire=False)


def _build():
    mesh = plsc.VectorSubcoreMesh(core_axis_name="c", subcore_axis_name="s")
    return pl.kernel(
        _sc_body,
        out_type=jax.ShapeDtypeStruct((_B, _C, _H, _W), jnp.float32),
        mesh=mesh,
        scratch_types=[
            pltpu.VMEM((_H // 8, 8, 128), jnp.float32),
            pltpu.VMEM((_H // 8, 8, 96), jnp.float32),
            pltpu.VMEM((_H // 8, 8, 128), jnp.float32),
            pltpu.VMEM((_H // 8, 8, 96), jnp.float32),
            pltpu.VMEM((4096,), jnp.int32),
            pltpu.VMEM((256,), jnp.int32),
            pltpu.SemaphoreType.DMA,
            pltpu.SemaphoreType.DMA,
            pltpu.SemaphoreType.DMA,
            pltpu.SemaphoreType.DMA,
        ],
        compiler_params=pltpu.CompilerParams(
            needs_layout_passes=False,
            use_tc_tiling_on_sc=True,
        ),
    )


def kernel(x):
    return _build()(x)


# speedup vs baseline: 2.9597x; 1.1523x over previous
import numpy as np
import jax
import jax.numpy as jnp
from jax import lax
from jax.experimental import pallas as pl
from jax.experimental.pallas import tpu as pltpu
from jax.experimental.pallas import tpu_sc as plsc

_B, _C, _H, _W = 8, 96, 224, 224
_N = _H * _W
_R = _B * _C
_K = int(0.3 * _N)
_M31 = np.int32(0x7FFFFFFF)
_NT = (_H // 8) * 2


def _f2key(v):
    u = plsc.bitcast(v, jnp.int32)
    return u ^ ((u >> 31) & _M31)


def _popcount(m):
    return jnp.max(plsc.all_reduce_population_count(m))


def _walk(histref, nvec, kr):
    lanes = lax.iota(jnp.int32, 16)

    def body(j, carry):
        found, dstar, newk, running = carry
        jj = nvec - 1 - j
        h = histref[pl.ds(jj * 16, 16)]
        suf = lax.rev(jnp.cumsum(lax.rev(h, (0,)), axis=0), (0,)) + running
        mask = suf >= kr
        c = _popcount(mask)
        has = jnp.logical_and(found == 0, c > 0)
        sel = lanes == (c - 1)
        s_d = jnp.max(jnp.where(sel, suf, 0))
        h_d = jnp.max(jnp.where(sel, h, 0))
        dstar = jnp.where(has, jj * 16 + c - 1, dstar)
        newk = jnp.where(has, kr - (s_d - h_d), newk)
        found = jnp.where(has, jnp.int32(1), found)
        running = jnp.max(suf)
        return found, dstar, newk, running

    z = jnp.int32(0)
    _, dstar, newk, _ = lax.fori_loop(0, nvec, body, (z, z, z, z))
    return dstar, newk


def _find_hier(hist, histc, nb, kr):
    lanes = lax.iota(jnp.int32, 16)
    dc, kr2 = _walk(histc, nb // 256, kr)
    h = hist[pl.ds(dc * 16, 16)]
    suf = lax.rev(jnp.cumsum(lax.rev(h, (0,)), axis=0), (0,))
    mask = suf >= kr2
    c = _popcount(mask)
    sel = lanes == (c - 1)
    s_d = jnp.max(jnp.where(sel, suf, 0))
    h_d = jnp.max(jnp.where(sel, h, 0))
    return dc * 16 + c - 1, kr2 - (s_d - h_d)


def _zero_hist(hist, histc, nb):
    zeros16 = jnp.zeros((16,), jnp.int32)

    @plsc.parallel_loop(0, nb, 16, unroll=4)
    def _(i):
        hist[pl.ds(i, 16)] = zeros16

    @plsc.parallel_loop(0, nb // 16, 16, unroll=1)
    def _(i):
        histc[pl.ds(i, 16)] = zeros16


def _build_coarse(hist, histc, nb):
    @plsc.parallel_loop(0, nb // 16, 1, unroll=4)
    def _(i):
        idx = jnp.full((16,), i, jnp.int32)
        plsc.addupdate_scatter(histc, [idx], hist[pl.ds(i * 16, 16)])


def _slices(dataA, dataB, rr2):
    tr = rr2 >> 3
    rs = rr2 & 7
    out = [(dataA, tr, rs, c * 16) for c in range(8)]
    out += [(dataB, tr, rs, c * 16) for c in range(6)]
    return out


def _row_threshold(dataA, dataB, hist, histc):
    lanes = lax.iota(jnp.int32, 16)
    ones16 = jnp.ones((16,), jnp.int32)

    _zero_hist(hist, histc, 4096)

    @plsc.parallel_loop(0, _H, 1, unroll=2)
    def _(r):
        for (ref, tr, rs, c) in _slices(dataA, dataB, r):
            key = _f2key(ref[tr, rs, pl.ds(c, 16)])
            d = (key >> 20) + 2048
            plsc.addupdate_scatter(hist, [d], ones16)

    _build_coarse(hist, histc, 4096)
    d1, kr1 = _find_hier(hist, histc, 4096, jnp.int32(_K))

    d1top = d1 - 2048
    _zero_hist(hist, histc, 1024)

    @plsc.parallel_loop(0, _H, 1, unroll=2)
    def _(r):
        for (ref, tr, rs, c) in _slices(dataA, dataB, r):
            key = _f2key(ref[tr, rs, pl.ds(c, 16)])
            t = key >> 10
            plsc.addupdate_scatter(hist, [t & jnp.int32(0x3FF)], ones16,
                                   mask=(t >> 10) == d1top)

    _build_coarse(hist, histc, 1024)
    d2, kr2 = _find_hier(hist, histc, 1024, kr1)

    _zero_hist(hist, histc, 1024)

    pre2 = (d1top << 10) | d2

    @plsc.parallel_loop(0, _H, 1, unroll=2)
    def _(r):
        for (ref, tr, rs, c) in _slices(dataA, dataB, r):
            key = _f2key(ref[tr, rs, pl.ds(c, 16)])
            plsc.addupdate_scatter(hist, [key & jnp.int32(0x3FF)], ones16,
                                   mask=(key >> 10) == pre2)

    _build_coarse(hist, histc, 1024)
    d3, _ = _find_hier(hist, histc, 1024, kr2)
    low20 = (d2 << 10) | d3

    thr_key = ((d1 - 2048) << 20) | low20
    tk = jnp.full((16,), thr_key, jnp.int32)
    return plsc.bitcast(tk ^ ((tk >> 31) & _M31), jnp.float32)


def _dma_map(x_hbm, b0, ch, dataA, dataB, sem, to_vmem, fire=True, drain=True):
    phases = ([0] if fire else []) + ([1] if drain else [])
    for phase in phases:
        for tr in range(_H // 8):
            for (ref, cs, cw) in ((dataA, 0, 128), (dataB, 128, 96)):
                hslice = x_hbm.at[b0, ch, pl.ds(tr * 8, 8), pl.ds(cs, cw)]
                vslice = ref.at[tr]
                src, dst = (hslice, vslice) if to_vmem else (vslice, hslice)
                if phase == 0:
                    pltpu.async_copy(src, dst, sem)
                else:
                    pltpu.make_async_copy(src, dst, sem).wait()


def _sc_body(x_hbm, out_hbm, dataA0, dataB0, dataA1, dataB1,
             hist, histc, isem0, isem1, osem0, osem1):
    nc = 2
    rpw = _R // (nc * 16)
    wid = lax.axis_index("s") * nc + lax.axis_index("c")
    b0 = wid // 4
    c0 = (wid % 4) * rpw
    zf16 = jnp.zeros((16,), jnp.float32)
    bufs = ((dataA0, dataB0), (dataA1, dataB1))
    isems = (isem0, isem1)
    osems = (osem0, osem1)

    _dma_map(x_hbm, b0, c0, dataA0, dataB0, isem0, True, drain=False)

    def pair_body(p, _):
        for b in range(2):
            rr = 2 * p + b
            ch = c0 + rr
            dA, dB = bufs[b]
            dAo, dBo = bufs[1 - b]

            _dma_map(x_hbm, b0, ch, dA, dB, isems[b], True, fire=False)

            thr = _row_threshold(dA, dB, hist, histc)

            @pl.when(rr > 0)
            def _():
                _dma_map(out_hbm, b0, ch - 1, dAo, dBo, osems[1 - b], False,
                         fire=False)

            @pl.when(rr + 1 < rpw)
            def _():
                _dma_map(x_hbm, b0, ch + 1, dAo, dBo, isems[1 - b], True,
                         drain=False)

            @plsc.parallel_loop(0, _H, 1, unroll=2)
            def _(r):
                for (ref, tr, rs, c) in _slices(dA, dB, r):
                    v = ref[tr, rs, pl.ds(c, 16)]
                    ref[tr, rs, pl.ds(c, 16)] = jnp.where(v >= thr, v, zf16)

            _dma_map(out_hbm, b0, ch, dA, dB, osems[b], False, drain=False)
        return 0

    lax.fori_loop(0, rpw // 2, pair_body, 0)
    _dma_map(out_hbm, b0, c0 + rpw - 1, dataA1, dataB1, osems[1], False,
             fire=False)


def _build():
    mesh = plsc.VectorSubcoreMesh(core_axis_name="c", subcore_axis_name="s")
    return pl.kernel(
        _sc_body,
        out_type=jax.ShapeDtypeStruct((_B, _C, _H, _W), jnp.float32),
        mesh=mesh,
        scratch_types=[
            pltpu.VMEM((_H // 8, 8, 128), jnp.float32),
            pltpu.VMEM((_H // 8, 8, 96), jnp.float32),
            pltpu.VMEM((_H // 8, 8, 128), jnp.float32),
            pltpu.VMEM((_H // 8, 8, 96), jnp.float32),
            pltpu.VMEM((4096,), jnp.int32),
            pltpu.VMEM((256,), jnp.int32),
            pltpu.SemaphoreType.DMA,
            pltpu.SemaphoreType.DMA,
            pltpu.SemaphoreType.DMA,
            pltpu.SemaphoreType.DMA,
        ],
        compiler_params=pltpu.CompilerParams(
            needs_layout_passes=False,
            use_tc_tiling_on_sc=True,
        ),
    )


def kernel(x):
    return _build()(x)


# speedup vs baseline: 3.2910x; 1.1120x over previous
import numpy as np
import jax
import jax.numpy as jnp
from jax import lax
from jax.experimental import pallas as pl
from jax.experimental.pallas import tpu as pltpu
from jax.experimental.pallas import tpu_sc as plsc

_B, _C, _H, _W = 8, 96, 224, 224
_N = _H * _W
_R = _B * _C
_K = int(0.3 * _N)
_M31 = np.int32(0x7FFFFFFF)
_NT = (_H // 8) * 2


def _f2key(v):
    u = plsc.bitcast(v, jnp.int32)
    return u ^ ((u >> 31) & _M31)


def _popcount(m):
    return jnp.max(plsc.all_reduce_population_count(m))


def _walk(histref, nvec, kr):
    lanes = lax.iota(jnp.int32, 16)

    def body(j, carry):
        found, dstar, newk, running = carry
        jj = nvec - 1 - j
        h = histref[pl.ds(jj * 16, 16)]
        suf = lax.rev(jnp.cumsum(lax.rev(h, (0,)), axis=0), (0,)) + running
        mask = suf >= kr
        c = _popcount(mask)
        has = jnp.logical_and(found == 0, c > 0)
        sel = lanes == (c - 1)
        s_d = jnp.max(jnp.where(sel, suf, 0))
        h_d = jnp.max(jnp.where(sel, h, 0))
        dstar = jnp.where(has, jj * 16 + c - 1, dstar)
        newk = jnp.where(has, kr - (s_d - h_d), newk)
        found = jnp.where(has, jnp.int32(1), found)
        running = jnp.max(suf)
        return found, dstar, newk, running

    z = jnp.int32(0)
    _, dstar, newk, _ = lax.fori_loop(0, nvec, body, (z, z, z, z))
    return dstar, newk


def _find_hier(hist, histc, nb, kr):
    lanes = lax.iota(jnp.int32, 16)
    dc, kr2 = _walk(histc, nb // 256, kr)
    h = hist[pl.ds(dc * 16, 16)]
    suf = lax.rev(jnp.cumsum(lax.rev(h, (0,)), axis=0), (0,))
    mask = suf >= kr2
    c = _popcount(mask)
    sel = lanes == (c - 1)
    s_d = jnp.max(jnp.where(sel, suf, 0))
    h_d = jnp.max(jnp.where(sel, h, 0))
    return dc * 16 + c - 1, kr2 - (s_d - h_d), h_d


def _zero_hist(hist, histc, nb):
    zeros16 = jnp.zeros((16,), jnp.int32)

    @plsc.parallel_loop(0, nb, 16, unroll=4)
    def _(i):
        hist[pl.ds(i, 16)] = zeros16

    @plsc.parallel_loop(0, nb // 16, 16, unroll=1)
    def _(i):
        histc[pl.ds(i, 16)] = zeros16


def _build_coarse(hist, histc, nb):
    @plsc.parallel_loop(0, nb // 16, 1, unroll=4)
    def _(i):
        idx = jnp.full((16,), i, jnp.int32)
        plsc.addupdate_scatter(histc, [idx], hist[pl.ds(i * 16, 16)])


def _slices(dataA, dataB, rr2):
    tr = rr2 >> 3
    rs = rr2 & 7
    out = [(dataA, tr, rs, c * 16) for c in range(8)]
    out += [(dataB, tr, rs, c * 16) for c in range(6)]
    return out


def _row_threshold(dataA, dataB, hist, histc):
    lanes = lax.iota(jnp.int32, 16)
    ones16 = jnp.ones((16,), jnp.int32)

    _zero_hist(hist, histc, 4096)

    @plsc.parallel_loop(0, _H, 1, unroll=2)
    def _(r):
        for (ref, tr, rs, c) in _slices(dataA, dataB, r):
            key = _f2key(ref[tr, rs, pl.ds(c, 16)])
            d = (key >> 20) + 2048
            plsc.addupdate_scatter(hist, [d], ones16)

    _build_coarse(hist, histc, 4096)
    d1, kr1, _ = _find_hier(hist, histc, 4096, jnp.int32(_K))

    d1top = d1 - 2048
    _zero_hist(hist, histc, 1024)

    @plsc.parallel_loop(0, _H, 1, unroll=2)
    def _(r):
        for (ref, tr, rs, c) in _slices(dataA, dataB, r):
            key = _f2key(ref[tr, rs, pl.ds(c, 16)])
            t = key >> 10
            plsc.addupdate_scatter(hist, [t & jnp.int32(0x3FF)], ones16,
                                   mask=(t >> 10) == d1top)

    _build_coarse(hist, histc, 1024)
    d2, kr2, c2 = _find_hier(hist, histc, 1024, kr1)

    pre2 = (d1top << 10) | d2

    def lvl3():
        _zero_hist(hist, histc, 1024)

        @plsc.parallel_loop(0, _H, 1, unroll=2)
        def _(r):
            for (ref, tr, rs, c) in _slices(dataA, dataB, r):
                key = _f2key(ref[tr, rs, pl.ds(c, 16)])
                plsc.addupdate_scatter(hist, [key & jnp.int32(0x3FF)], ones16,
                                       mask=(key >> 10) == pre2)

        _build_coarse(hist, histc, 1024)
        d3, _, _ = _find_hier(hist, histc, 1024, kr2)
        return (d2 << 10) | d3

    low20 = lax.cond(kr2 == c2, lambda: d2 << 10, lvl3)
    thr_key = ((d1 - 2048) << 20) | low20
    tk = jnp.full((16,), thr_key, jnp.int32)
    return plsc.bitcast(tk ^ ((tk >> 31) & _M31), jnp.float32)


def _dma_map(x_hbm, b0, ch, dataA, dataB, sem, to_vmem, fire=True, drain=True):
    phases = ([0] if fire else []) + ([1] if drain else [])
    for phase in phases:
        for tr in range(_H // 8):
            for (ref, cs, cw) in ((dataA, 0, 128), (dataB, 128, 96)):
                hslice = x_hbm.at[b0, ch, pl.ds(tr * 8, 8), pl.ds(cs, cw)]
                vslice = ref.at[tr]
                src, dst = (hslice, vslice) if to_vmem else (vslice, hslice)
                if phase == 0:
                    pltpu.async_copy(src, dst, sem)
                else:
                    pltpu.make_async_copy(src, dst, sem).wait()


def _sc_body(x_hbm, out_hbm, dataA0, dataB0, dataA1, dataB1,
             hist, histc, isem0, isem1, osem0, osem1):
    nc = 2
    rpw = _R // (nc * 16)
    wid = lax.axis_index("s") * nc + lax.axis_index("c")
    b0 = wid // 4
    c0 = (wid % 4) * rpw
    zf16 = jnp.zeros((16,), jnp.float32)
    bufs = ((dataA0, dataB0), (dataA1, dataB1))
    isems = (isem0, isem1)
    osems = (osem0, osem1)

    _dma_map(x_hbm, b0, c0, dataA0, dataB0, isem0, True, drain=False)

    def pair_body(p, _):
        for b in range(2):
            rr = 2 * p + b
            ch = c0 + rr
            dA, dB = bufs[b]
            dAo, dBo = bufs[1 - b]

            _dma_map(x_hbm, b0, ch, dA, dB, isems[b], True, fire=False)

            thr = _row_threshold(dA, dB, hist, histc)

            @pl.when(rr > 0)
            def _():
                _dma_map(out_hbm, b0, ch - 1, dAo, dBo, osems[1 - b], False,
                         fire=False)

            @pl.when(rr + 1 < rpw)
            def _():
                _dma_map(x_hbm, b0, ch + 1, dAo, dBo, isems[1 - b], True,
                         drain=False)

            @plsc.parallel_loop(0, _H, 1, unroll=2)
            def _(r):
                for (ref, tr, rs, c) in _slices(dA, dB, r):
                    v = ref[tr, rs, pl.ds(c, 16)]
                    ref[tr, rs, pl.ds(c, 16)] = jnp.where(v >= thr, v, zf16)

            _dma_map(out_hbm, b0, ch, dA, dB, osems[b], False, drain=False)
        return 0

    lax.fori_loop(0, rpw // 2, pair_body, 0)
    _dma_map(out_hbm, b0, c0 + rpw - 1, dataA1, dataB1, osems[1], False,
             fire=False)


def _build():
    mesh = plsc.VectorSubcoreMesh(core_axis_name="c", subcore_axis_name="s")
    return pl.kernel(
        _sc_body,
        out_type=jax.ShapeDtypeStruct((_B, _C, _H, _W), jnp.float32),
        mesh=mesh,
        scratch_types=[
            pltpu.VMEM((_H // 8, 8, 128), jnp.float32),
            pltpu.VMEM((_H // 8, 8, 96), jnp.float32),
            pltpu.VMEM((_H // 8, 8, 128), jnp.float32),
            pltpu.VMEM((_H // 8, 8, 96), jnp.float32),
            pltpu.VMEM((4096,), jnp.int32),
            pltpu.VMEM((256,), jnp.int32),
            pltpu.SemaphoreType.DMA,
            pltpu.SemaphoreType.DMA,
            pltpu.SemaphoreType.DMA,
            pltpu.SemaphoreType.DMA,
        ],
        compiler_params=pltpu.CompilerParams(
            needs_layout_passes=False,
            use_tc_tiling_on_sc=True,
        ),
    )


def kernel(x):
    return _build()(x)


# speedup vs baseline: 3.3030x; 1.0036x over previous
import numpy as np
import jax
import jax.numpy as jnp
from jax import lax
from jax.experimental import pallas as pl
from jax.experimental.pallas import tpu as pltpu
from jax.experimental.pallas import tpu_sc as plsc

_B, _C, _H, _W = 8, 96, 224, 224
_N = _H * _W
_R = _B * _C
_K = int(0.3 * _N)
_M31 = np.int32(0x7FFFFFFF)
_NT = (_H // 8) * 2


def _f2key(v):
    u = plsc.bitcast(v, jnp.int32)
    return u ^ ((u >> 31) & _M31)


def _popcount(m):
    return jnp.max(plsc.all_reduce_population_count(m))


def _walk(histref, nvec, kr):
    lanes = lax.iota(jnp.int32, 16)

    def body(j, carry):
        found, dstar, newk, running = carry
        jj = nvec - 1 - j
        h = histref[pl.ds(jj * 16, 16)]
        suf = lax.rev(jnp.cumsum(lax.rev(h, (0,)), axis=0), (0,)) + running
        mask = suf >= kr
        c = _popcount(mask)
        has = jnp.logical_and(found == 0, c > 0)
        sel = lanes == (c - 1)
        s_d = jnp.max(jnp.where(sel, suf, 0))
        h_d = jnp.max(jnp.where(sel, h, 0))
        dstar = jnp.where(has, jj * 16 + c - 1, dstar)
        newk = jnp.where(has, kr - (s_d - h_d), newk)
        found = jnp.where(has, jnp.int32(1), found)
        running = jnp.max(suf)
        return found, dstar, newk, running

    z = jnp.int32(0)
    _, dstar, newk, _ = lax.fori_loop(0, nvec, body, (z, z, z, z))
    return dstar, newk


def _find_hier(hist, histc, nb, kr):
    lanes = lax.iota(jnp.int32, 16)
    dc, kr2 = _walk(histc, nb // 256, kr)
    h = hist[pl.ds(dc * 16, 16)]
    suf = lax.rev(jnp.cumsum(lax.rev(h, (0,)), axis=0), (0,))
    mask = suf >= kr2
    c = _popcount(mask)
    sel = lanes == (c - 1)
    s_d = jnp.max(jnp.where(sel, suf, 0))
    h_d = jnp.max(jnp.where(sel, h, 0))
    return dc * 16 + c - 1, kr2 - (s_d - h_d), h_d


def _zero_hist(hist, histc, nb):
    zeros16 = jnp.zeros((16,), jnp.int32)

    @plsc.parallel_loop(0, nb, 16, unroll=4)
    def _(i):
        hist[pl.ds(i, 16)] = zeros16

    @plsc.parallel_loop(0, nb // 16, 16, unroll=1)
    def _(i):
        histc[pl.ds(i, 16)] = zeros16


def _build_coarse(hist, histc, nb):
    @plsc.parallel_loop(0, nb // 16, 1, unroll=4)
    def _(i):
        idx = jnp.full((16,), i, jnp.int32)
        plsc.addupdate_scatter(histc, [idx], hist[pl.ds(i * 16, 16)])


def _slices(dataA, dataB, rr2):
    tr = rr2 >> 3
    rs = rr2 & 7
    out = [(dataA, tr, rs, c * 16) for c in range(8)]
    out += [(dataB, tr, rs, c * 16) for c in range(6)]
    return out


def _row_threshold(dataA, dataB, hist, histc):
    lanes = lax.iota(jnp.int32, 16)
    ones16 = jnp.ones((16,), jnp.int32)

    _zero_hist(hist, histc, 4096)

    @plsc.parallel_loop(0, _H, 1, unroll=4)
    def _(r):
        for (ref, tr, rs, c) in _slices(dataA, dataB, r):
            key = _f2key(ref[tr, rs, pl.ds(c, 16)])
            d = (key >> 20) + 2048
            plsc.addupdate_scatter(hist, [d], ones16)

    _build_coarse(hist, histc, 4096)
    d1, kr1, _ = _find_hier(hist, histc, 4096, jnp.int32(_K))

    d1top = d1 - 2048
    _zero_hist(hist, histc, 1024)

    @plsc.parallel_loop(0, _H, 1, unroll=4)
    def _(r):
        for (ref, tr, rs, c) in _slices(dataA, dataB, r):
            key = _f2key(ref[tr, rs, pl.ds(c, 16)])
            t = key >> 10
            plsc.addupdate_scatter(hist, [t & jnp.int32(0x3FF)], ones16,
                                   mask=(t >> 10) == d1top)

    _build_coarse(hist, histc, 1024)
    d2, kr2, c2 = _find_hier(hist, histc, 1024, kr1)

    pre2 = (d1top << 10) | d2

    def lvl3():
        _zero_hist(hist, histc, 1024)

        @plsc.parallel_loop(0, _H, 1, unroll=4)
        def _(r):
            for (ref, tr, rs, c) in _slices(dataA, dataB, r):
                key = _f2key(ref[tr, rs, pl.ds(c, 16)])
                plsc.addupdate_scatter(hist, [key & jnp.int32(0x3FF)], ones16,
                                       mask=(key >> 10) == pre2)

        _build_coarse(hist, histc, 1024)
        d3, _, _ = _find_hier(hist, histc, 1024, kr2)
        return (d2 << 10) | d3

    low20 = lax.cond(kr2 == c2, lambda: d2 << 10, lvl3)
    thr_key = ((d1 - 2048) << 20) | low20
    tk = jnp.full((16,), thr_key, jnp.int32)
    return plsc.bitcast(tk ^ ((tk >> 31) & _M31), jnp.float32)


def _dma_map(x_hbm, b0, ch, dataA, dataB, sem, to_vmem, fire=True, drain=True):
    phases = ([0] if fire else []) + ([1] if drain else [])
    for phase in phases:
        for tr in range(_H // 8):
            for (ref, cs, cw) in ((dataA, 0, 128), (dataB, 128, 96)):
                hslice = x_hbm.at[b0, ch, pl.ds(tr * 8, 8), pl.ds(cs, cw)]
                vslice = ref.at[tr]
                src, dst = (hslice, vslice) if to_vmem else (vslice, hslice)
                if phase == 0:
                    pltpu.async_copy(src, dst, sem)
                else:
                    pltpu.make_async_copy(src, dst, sem).wait()


def _sc_body(x_hbm, out_hbm, dataA0, dataB0, dataA1, dataB1,
             hist, histc, isem0, isem1, osem0, osem1):
    nc = 2
    rpw = _R // (nc * 16)
    wid = lax.axis_index("s") * nc + lax.axis_index("c")
    b0 = wid // 4
    c0 = (wid % 4) * rpw
    zf16 = jnp.zeros((16,), jnp.float32)
    bufs = ((dataA0, dataB0), (dataA1, dataB1))
    isems = (isem0, isem1)
    osems = (osem0, osem1)

    _dma_map(x_hbm, b0, c0, dataA0, dataB0, isem0, True, drain=False)

    def pair_body(p, _):
        for b in range(2):
            rr = 2 * p + b
            ch = c0 + rr
            dA, dB = bufs[b]
            dAo, dBo = bufs[1 - b]

            _dma_map(x_hbm, b0, ch, dA, dB, isems[b], True, fire=False)

            thr = _row_threshold(dA, dB, hist, histc)

            @pl.when(rr > 0)
            def _():
                _dma_map(out_hbm, b0, ch - 1, dAo, dBo, osems[1 - b], False,
                         fire=False)

            @pl.when(rr + 1 < rpw)
            def _():
                _dma_map(x_hbm, b0, ch + 1, dAo, dBo, isems[1 - b], True,
                         drain=False)

            @plsc.parallel_loop(0, _H, 1, unroll=4)
            def _(r):
                for (ref, tr, rs, c) in _slices(dA, dB, r):
                    v = ref[tr, rs, pl.ds(c, 16)]
                    ref[tr, rs, pl.ds(c, 16)] = jnp.where(v >= thr, v, zf16)

            _dma_map(out_hbm, b0, ch, dA, dB, osems[b], False, drain=False)
        return 0

    lax.fori_loop(0, rpw // 2, pair_body, 0)
    _dma_map(out_hbm, b0, c0 + rpw - 1, dataA1, dataB1, osems[1], False,
             fire=False)


def _build():
    mesh = plsc.VectorSubcoreMesh(core_axis_name="c", subcore_axis_name="s")
    return pl.kernel(
        _sc_body,
        out_type=jax.ShapeDtypeStruct((_B, _C, _H, _W), jnp.float32),
        mesh=mesh,
        scratch_types=[
            pltpu.VMEM((_H // 8, 8, 128), jnp.float32),
            pltpu.VMEM((_H // 8, 8, 96), jnp.float32),
            pltpu.VMEM((_H // 8, 8, 128), jnp.float32),
            pltpu.VMEM((_H // 8, 8, 96), jnp.float32),
            pltpu.VMEM((4096,), jnp.int32),
            pltpu.VMEM((256,), jnp.int32),
            pltpu.SemaphoreType.DMA,
            pltpu.SemaphoreType.DMA,
            pltpu.SemaphoreType.DMA,
            pltpu.SemaphoreType.DMA,
        ],
        compiler_params=pltpu.CompilerParams(
            needs_layout_passes=False,
            use_tc_tiling_on_sc=True,
        ),
    )


def kernel(x):
    return _build()(x)


# speedup vs baseline: 3.3438x; 1.0124x over previous
import numpy as np
import jax
import jax.numpy as jnp
from jax import lax
from jax.experimental import pallas as pl
from jax.experimental.pallas import tpu as pltpu
from jax.experimental.pallas import tpu_sc as plsc

_B, _C, _H, _W = 8, 96, 224, 224
_N = _H * _W
_R = _B * _C
_K = int(0.3 * _N)
_M31 = np.int32(0x7FFFFFFF)
_NT = (_H // 8) * 2


def _f2key(v):
    u = plsc.bitcast(v, jnp.int32)
    return u ^ ((u >> 31) & _M31)


def _popcount(m):
    return jnp.max(plsc.all_reduce_population_count(m))


def _walk(histref, nvec, kr):
    lanes = lax.iota(jnp.int32, 16)

    def body(j, carry):
        found, dstar, newk, running = carry
        jj = nvec - 1 - j
        h = histref[pl.ds(jj * 16, 16)]
        suf = lax.rev(jnp.cumsum(lax.rev(h, (0,)), axis=0), (0,)) + running
        mask = suf >= kr
        c = _popcount(mask)
        has = jnp.logical_and(found == 0, c > 0)
        sel = lanes == (c - 1)
        s_d = jnp.max(jnp.where(sel, suf, 0))
        h_d = jnp.max(jnp.where(sel, h, 0))
        dstar = jnp.where(has, jj * 16 + c - 1, dstar)
        newk = jnp.where(has, kr - (s_d - h_d), newk)
        found = jnp.where(has, jnp.int32(1), found)
        running = jnp.max(suf)
        return found, dstar, newk, running

    z = jnp.int32(0)
    _, dstar, newk, _ = lax.fori_loop(0, nvec, body, (z, z, z, z))
    return dstar, newk


def _find_hier(hist, histc, nb, kr):
    lanes = lax.iota(jnp.int32, 16)
    dc, kr2 = _walk(histc, nb // 256, kr)
    h = hist[pl.ds(dc * 16, 16)]
    suf = lax.rev(jnp.cumsum(lax.rev(h, (0,)), axis=0), (0,))
    mask = suf >= kr2
    c = _popcount(mask)
    sel = lanes == (c - 1)
    s_d = jnp.max(jnp.where(sel, suf, 0))
    h_d = jnp.max(jnp.where(sel, h, 0))
    return dc * 16 + c - 1, kr2 - (s_d - h_d), h_d


def _zero_hist(hist, histc, nb):
    zeros16 = jnp.zeros((16,), jnp.int32)

    @plsc.parallel_loop(0, nb, 16, unroll=4)
    def _(i):
        hist[pl.ds(i, 16)] = zeros16

    @plsc.parallel_loop(0, nb // 16, 16, unroll=1)
    def _(i):
        histc[pl.ds(i, 16)] = zeros16


def _build_coarse(hist, histc, nb):
    @plsc.parallel_loop(0, nb // 16, 1, unroll=4)
    def _(i):
        idx = jnp.full((16,), i, jnp.int32)
        plsc.addupdate_scatter(histc, [idx], hist[pl.ds(i * 16, 16)])


def _slices(dataA, dataB, rr2):
    tr = rr2 >> 3
    rs = rr2 & 7
    out = [(dataA, tr, rs, c * 16) for c in range(8)]
    out += [(dataB, tr, rs, c * 16) for c in range(6)]
    return out


def _row_threshold(dataA, dataB, hist, histc, mid_cb):
    lanes = lax.iota(jnp.int32, 16)
    ones16 = jnp.ones((16,), jnp.int32)

    _zero_hist(hist, histc, 4096)

    @plsc.parallel_loop(0, _H, 1, unroll=4)
    def _(r):
        for (ref, tr, rs, c) in _slices(dataA, dataB, r):
            key = _f2key(ref[tr, rs, pl.ds(c, 16)])
            d = (key >> 20) + 2048
            plsc.addupdate_scatter(hist, [d], ones16)

    mid_cb()
    _build_coarse(hist, histc, 4096)
    d1, kr1, _ = _find_hier(hist, histc, 4096, jnp.int32(_K))

    d1top = d1 - 2048
    _zero_hist(hist, histc, 1024)

    @plsc.parallel_loop(0, _H, 1, unroll=4)
    def _(r):
        for (ref, tr, rs, c) in _slices(dataA, dataB, r):
            key = _f2key(ref[tr, rs, pl.ds(c, 16)])
            t = key >> 10
            plsc.addupdate_scatter(hist, [t & jnp.int32(0x3FF)], ones16,
                                   mask=(t >> 10) == d1top)

    _build_coarse(hist, histc, 1024)
    d2, kr2, c2 = _find_hier(hist, histc, 1024, kr1)

    pre2 = (d1top << 10) | d2

    def lvl3():
        _zero_hist(hist, histc, 1024)

        @plsc.parallel_loop(0, _H, 1, unroll=4)
        def _(r):
            for (ref, tr, rs, c) in _slices(dataA, dataB, r):
                key = _f2key(ref[tr, rs, pl.ds(c, 16)])
                plsc.addupdate_scatter(hist, [key & jnp.int32(0x3FF)], ones16,
                                       mask=(key >> 10) == pre2)

        _build_coarse(hist, histc, 1024)
        d3, _, _ = _find_hier(hist, histc, 1024, kr2)
        return (d2 << 10) | d3

    low20 = lax.cond(kr2 == c2, lambda: d2 << 10, lvl3)
    thr_key = ((d1 - 2048) << 20) | low20
    tk = jnp.full((16,), thr_key, jnp.int32)
    return plsc.bitcast(tk ^ ((tk >> 31) & _M31), jnp.float32)


def _dma_map(x_hbm, b0, ch, dataA, dataB, sem, to_vmem, fire=True, drain=True):
    phases = ([0] if fire else []) + ([1] if drain else [])
    for phase in phases:
        for tr in range(_H // 8):
            for (ref, cs, cw) in ((dataA, 0, 128), (dataB, 128, 96)):
                hslice = x_hbm.at[b0, ch, pl.ds(tr * 8, 8), pl.ds(cs, cw)]
                vslice = ref.at[tr]
                src, dst = (hslice, vslice) if to_vmem else (vslice, hslice)
                if phase == 0:
                    pltpu.async_copy(src, dst, sem)
                else:
                    pltpu.make_async_copy(src, dst, sem).wait()


def _sc_body(x_hbm, out_hbm, dataA0, dataB0, dataA1, dataB1,
             hist, histc, isem0, isem1, osem0, osem1):
    nc = 2
    rpw = _R // (nc * 16)
    wid = lax.axis_index("s") * nc + lax.axis_index("c")
    b0 = wid // 4
    c0 = (wid % 4) * rpw
    zf16 = jnp.zeros((16,), jnp.float32)
    bufs = ((dataA0, dataB0), (dataA1, dataB1))
    isems = (isem0, isem1)
    osems = (osem0, osem1)

    _dma_map(x_hbm, b0, c0, dataA0, dataB0, isem0, True, drain=False)

    def pair_body(p, _):
        for b in range(2):
            rr = 2 * p + b
            ch = c0 + rr
            dA, dB = bufs[b]
            dAo, dBo = bufs[1 - b]

            _dma_map(x_hbm, b0, ch, dA, dB, isems[b], True, fire=False)

            def prefetch():
                @pl.when(rr > 0)
                def _():
                    _dma_map(out_hbm, b0, ch - 1, dAo, dBo, osems[1 - b],
                             False, fire=False)

                @pl.when(rr + 1 < rpw)
                def _():
                    _dma_map(x_hbm, b0, ch + 1, dAo, dBo, isems[1 - b], True,
                             drain=False)

            thr = _row_threshold(dA, dB, hist, histc, prefetch)

            @plsc.parallel_loop(0, _H, 1, unroll=4)
            def _(r):
                for (ref, tr, rs, c) in _slices(dA, dB, r):
                    v = ref[tr, rs, pl.ds(c, 16)]
                    ref[tr, rs, pl.ds(c, 16)] = jnp.where(v >= thr, v, zf16)

            _dma_map(out_hbm, b0, ch, dA, dB, osems[b], False, drain=False)
        return 0

    lax.fori_loop(0, rpw // 2, pair_body, 0)
    _dma_map(out_hbm, b0, c0 + rpw - 1, dataA1, dataB1, osems[1], False,
             fire=False)


def _build():
    mesh = plsc.VectorSubcoreMesh(core_axis_name="c", subcore_axis_name="s")
    return pl.kernel(
        _sc_body,
        out_type=jax.ShapeDtypeStruct((_B, _C, _H, _W), jnp.float32),
        mesh=mesh,
        scratch_types=[
            pltpu.VMEM((_H // 8, 8, 128), jnp.float32),
            pltpu.VMEM((_H // 8, 8, 96), jnp.float32),
            pltpu.VMEM((_H // 8, 8, 128), jnp.float32),
            pltpu.VMEM((_H // 8, 8, 96), jnp.float32),
            pltpu.VMEM((4096,), jnp.int32),
            pltpu.VMEM((256,), jnp.int32),
            pltpu.SemaphoreType.DMA,
            pltpu.SemaphoreType.DMA,
            pltpu.SemaphoreType.DMA,
            pltpu.SemaphoreType.DMA,
        ],
        compiler_params=pltpu.CompilerParams(
            needs_layout_passes=False,
            use_tc_tiling_on_sc=True,
        ),
    )


def kernel(x):
    return _build()(x)
